# baseline (speedup 1.0000x reference)
import functools

import jax
import jax.numpy as jnp
from jax import lax
from jax.experimental import pallas as pl
from jax.experimental.pallas import tpu as pltpu
from jax.experimental.pallas import tpu_sc as plsc

_VOCAB = 1000000
_BATCH = 16384
_DIM = 32
_LANES = 128
_RPG = _LANES // _DIM

_NC = 2
_NS = 16
_NW = _NC * _NS
_BPW = _BATCH // _NW
_OPW = _BPW // _RPG


@jax.jit
def kernel(indices, in_vec):
    table128 = in_vec.reshape(_VOCAB // _RPG, _LANES)
    mesh = plsc.VectorSubcoreMesh(core_axis_name="c", subcore_axis_name="s")

    @functools.partial(
        pl.kernel,
        mesh=mesh,
        out_type=jax.ShapeDtypeStruct((_BATCH // _RPG, _LANES), jnp.float32),
        scratch_types=[
            pltpu.VMEM((_BPW,), jnp.int32),
            pltpu.VMEM((_BPW,), jnp.int32),
            pltpu.VMEM((_BPW, _LANES), jnp.float32),
            pltpu.VMEM((_OPW, _LANES), jnp.float32),
            pltpu.SemaphoreType.DMA,
        ],
        compiler_params=pltpu.CompilerParams(needs_layout_passes=False),
    )
    def gather_kernel(idx_hbm, table_hbm, out_hbm, idx_v, grp_v, rows_v,
                      out_v, sem):
        wid = lax.axis_index("s") * _NC + lax.axis_index("c")
        base = wid * _BPW
        pltpu.sync_copy(idx_hbm.at[pl.ds(base, _BPW)], idx_v)

        def grp_body(g, carry):
            iv = idx_v[pl.ds(g * 16, 16)]
            grp_v[pl.ds(g * 16, 16)] = lax.shift_right_logical(iv, 2)
            return carry

        lax.fori_loop(0, _BPW // 16, grp_body, 0)

        pltpu.async_copy(table_hbm.at[grp_v], rows_v, sem).wait()

        def extract_body(g, carry):
            lane = lax.iota(jnp.int32, 16)
            r = lane + g * 16
            iv = idx_v[pl.ds(g * 16, 16)]
            col0 = (iv & (_RPG - 1)) * _DIM
            orow = lax.shift_right_logical(r, 2)
            ocol0 = (r & (_RPG - 1)) * _DIM
            for j in range(_DIM):
                v = plsc.load_gather(rows_v, [r, col0 + j])
                plsc.store_scatter(out_v, [orow, ocol0 + j], v)
            return carry

        lax.fori_loop(0, _BPW // 16, extract_body, 0)

        pltpu.sync_copy(out_v, out_hbm.at[pl.ds(wid * _OPW, _OPW)])

    out = gather_kernel(indices.astype(jnp.int32), table128)
    return out.reshape(_BATCH, _DIM)


# speedup vs baseline: 3.7289x; 3.7289x over previous
import functools

import jax
import jax.numpy as jnp
from jax import lax
from jax.experimental import pallas as pl
from jax.experimental.pallas import tpu as pltpu
from jax.experimental.pallas import tpu_sc as plsc

_VOCAB = 1000000
_BATCH = 16384
_DIM = 32

_NC = 2
_NS = 16
_NW = _NC * _NS
_BPW = _BATCH // _NW
_CHUNK = 16


@jax.jit
def kernel(indices, in_vec):
    mesh = plsc.VectorSubcoreMesh(core_axis_name="c", subcore_axis_name="s")

    @functools.partial(
        pl.kernel,
        mesh=mesh,
        out_type=jax.ShapeDtypeStruct((_DIM, _BATCH), jnp.float32),
        scratch_types=[
            pltpu.VMEM((_BPW,), jnp.int32),
            pltpu.VMEM((_CHUNK, _DIM, 128), jnp.float32),
            pltpu.VMEM((_DIM, _BPW), jnp.float32),
            pltpu.SemaphoreType.DMA,
            pltpu.SemaphoreType.DMA,
        ],
        compiler_params=pltpu.CompilerParams(needs_layout_passes=False),
    )
    def gather_kernel(idx_hbm, table_hbm, out_hbm, idx_s, ring_v, block_v,
                      sem_i, sem_g):
        wid = lax.axis_index("s") * _NC + lax.axis_index("c")
        base = wid * _BPW
        pltpu.sync_copy(idx_hbm.at[pl.ds(base, _BPW)], idx_s)

        rows_lo = lax.iota(jnp.int32, 16)
        rows_hi = rows_lo + 16

        def chunk_body(ck, carry):
            iv = idx_s[pl.ds(ck * _CHUNK, _CHUNK)]
            for j in range(_CHUNK):
                col0 = pl.multiple_of(
                    lax.shift_right_logical(iv[j], 7) * 128, 128
                )
                pltpu.make_async_copy(
                    table_hbm.at[:, pl.ds(col0, 128)],
                    ring_v.at[j],
                    sem_g,
                ).start()
            for j in range(_CHUNK):
                pltpu.make_async_copy(
                    table_hbm.at[:, pl.ds(0, 128)], ring_v.at[j], sem_g
                ).wait()
            for j in range(_CHUNK):
                col = jnp.full((16,), iv[j] & 127, dtype=jnp.int32)
                pos = jnp.full((16,), ck * _CHUNK + j, dtype=jnp.int32)
                v_lo = plsc.load_gather(ring_v.at[j], [rows_lo, col])
                v_hi = plsc.load_gather(ring_v.at[j], [rows_hi, col])
                plsc.store_scatter(block_v, [rows_lo, pos], v_lo)
                plsc.store_scatter(block_v, [rows_hi, pos], v_hi)
            return carry

        lax.fori_loop(0, _BPW // _CHUNK, chunk_body, 0)

        pltpu.sync_copy(block_v, out_hbm.at[:, pl.ds(base, _BPW)])

    out_t = gather_kernel(indices.astype(jnp.int32), in_vec.T)
    return out_t.T
